# Initial kernel scaffold; baseline (speedup 1.0000x reference)
import jax
import jax.numpy as jnp
from jax.experimental import pallas as pl


def kernel(input, wg, We, be):
    raise NotImplementedError("write your pallas kernel here")


# speedup vs baseline: 4.1918x; 4.1918x over previous
import functools

import jax
import jax.numpy as jnp
from jax.experimental import pallas as pl
from jax.experimental.pallas import tpu as pltpu


def _wsum_body(we_ref, out_ref):
    k = pl.program_id(1)

    @pl.when(k == 0)
    def _init():
        out_ref[...] = jnp.zeros_like(out_ref)

    out_ref[0] += jnp.sum(we_ref[0], axis=0, keepdims=True)


def _main_body(x_ref, w4_ref, be_ref, sl_ref, out_ref, cnt_ref, *, cap, t):
    c = pl.program_id(0)

    @pl.when(c == 0)
    def _init():
        cnt_ref[0] = 0.0

    lt = jnp.dot(x_ref[...], w4_ref[...], preferred_element_type=jnp.float32)
    l0, l1, t0, t1 = lt[:, 0:1], lt[:, 1:2], lt[:, 2:3], lt[:, 3:4]
    e1 = l1 > l0
    m = e1.astype(jnp.float32)

    excl = jnp.dot(sl_ref[...], m, preferred_element_type=jnp.float32)
    pos1 = excl + cnt_ref[0]
    slin = (c * t + jax.lax.broadcasted_iota(jnp.int32, (t, 1), 0)).astype(
        jnp.float32)
    pos0 = slin - pos1
    pos = jnp.where(e1, pos1, pos0)
    within = (pos < cap).astype(jnp.float32)

    gate = jax.nn.sigmoid(jnp.abs(l1 - l0))
    bs0 = jnp.sum(be_ref[0:1, :])
    bs1 = jnp.sum(be_ref[1:2, :])
    tsel = jnp.where(e1, t1 + bs1, t0 + bs0)
    out_ref[...] = gate * within * tsel
    cnt_ref[0] += jnp.sum(m)


def _lsm_body(v_ref, out_ref):
    v = v_ref[...]
    mx = jnp.max(v, axis=1, keepdims=True)
    lse = jnp.log(jnp.sum(jnp.exp(v - mx), axis=1, keepdims=True)) + mx
    out_ref[...] = v - lse


def kernel(input, wg, We, be):
    B, SEQ, D = input.shape
    E = wg.shape[1]
    S = B * SEQ
    cap = -(-S // E)

    x = input.reshape(S, D)

    F = 256
    K = D // F
    wsum = pl.pallas_call(
        _wsum_body,
        grid=(E, K),
        in_specs=[pl.BlockSpec((1, F, D), lambda e, k: (e, k, 0))],
        out_specs=pl.BlockSpec((1, 1, D), lambda e, k: (e, 0, 0)),
        out_shape=jax.ShapeDtypeStruct((E, 1, D), jnp.float32),
    )(We)

    w4 = jnp.concatenate([wg, wsum.reshape(E, D).T], axis=1)

    T = 512
    C = S // T
    ii = jax.lax.broadcasted_iota(jnp.int32, (T, T), 0)
    jj = jax.lax.broadcasted_iota(jnp.int32, (T, T), 1)
    sl = (jj < ii).astype(jnp.float32)

    val = pl.pallas_call(
        functools.partial(_main_body, cap=float(cap), t=T),
        grid=(C,),
        in_specs=[
            pl.BlockSpec((T, D), lambda c: (c, 0)),
            pl.BlockSpec((D, 4), lambda c: (0, 0)),
            pl.BlockSpec((E, D), lambda c: (0, 0)),
            pl.BlockSpec((T, T), lambda c: (0, 0)),
        ],
        out_specs=pl.BlockSpec((T, 1), lambda c: (c, 0)),
        out_shape=jax.ShapeDtypeStruct((S, 1), jnp.float32),
        scratch_shapes=[pltpu.SMEM((1,), jnp.float32)],
    )(x, w4, be, sl)

    v = val.reshape(B, SEQ)

    out = pl.pallas_call(
        _lsm_body,
        in_specs=[pl.BlockSpec((B, SEQ), lambda: (0, 0))],
        out_specs=pl.BlockSpec((B, SEQ), lambda: (0, 0)),
        out_shape=jax.ShapeDtypeStruct((B, SEQ), jnp.float32),
    )(v)
    return out


# speedup vs baseline: 4.7167x; 1.1252x over previous
import functools

import jax
import jax.numpy as jnp
from jax.experimental import pallas as pl
from jax.experimental.pallas import tpu as pltpu


def _fused_body(we_ref, x_ref, wg_ref, be_ref, sl_ref, out_ref,
                w4_ref, cnt_ref, *, cap, t, kwe, kpe):
    k = pl.program_id(0)

    @pl.when(k == 0)
    def _init():
        w4_ref[0:2] = jnp.transpose(wg_ref[...])
        w4_ref[2:4] = jnp.zeros_like(w4_ref[2:4])
        cnt_ref[0] = 0.0

    @pl.when(k < kwe)
    def _accum():
        e = k // kpe
        part = jnp.sum(we_ref[0], axis=0, keepdims=True)
        w4_ref[pl.ds(2 + e, 1)] += part

    @pl.when(k >= kwe)
    def _tokens():
        c = k - kwe
        lt = jax.lax.dot_general(
            x_ref[...], w4_ref[...],
            dimension_numbers=(((1,), (1,)), ((), ())),
            preferred_element_type=jnp.float32)
        l0, l1, t0, t1 = lt[:, 0:1], lt[:, 1:2], lt[:, 2:3], lt[:, 3:4]
        e1 = l1 > l0
        m = e1.astype(jnp.float32)

        excl = jnp.dot(sl_ref[...], m, preferred_element_type=jnp.float32)
        pos1 = excl + cnt_ref[0]
        slin = (c * t + jax.lax.broadcasted_iota(jnp.int32, (t, 1), 0)
                ).astype(jnp.float32)
        pos0 = slin - pos1
        pos = jnp.where(e1, pos1, pos0)
        within = (pos < cap).astype(jnp.float32)

        gate = jax.nn.sigmoid(jnp.abs(l1 - l0))
        bs0 = jnp.sum(be_ref[0:1, :])
        bs1 = jnp.sum(be_ref[1:2, :])
        tsel = jnp.where(e1, t1 + bs1, t0 + bs0)
        out_ref[...] = gate * within * tsel
        cnt_ref[0] += jnp.sum(m)


def _lsm_body(v_ref, out_ref):
    v = v_ref[...]
    mx = jnp.max(v, axis=1, keepdims=True)
    lse = jnp.log(jnp.sum(jnp.exp(v - mx), axis=1, keepdims=True)) + mx
    out_ref[...] = v - lse


def kernel(input, wg, We, be):
    B, SEQ, D = input.shape
    E = wg.shape[1]
    S = B * SEQ
    cap = -(-S // E)

    x = input.reshape(S, D)

    F = 512
    KPE = D // F
    KWE = E * KPE
    T = 512
    C = S // T

    ii = jax.lax.broadcasted_iota(jnp.int32, (T, T), 0)
    jj = jax.lax.broadcasted_iota(jnp.int32, (T, T), 1)
    sl = (jj < ii).astype(jnp.float32)

    val = pl.pallas_call(
        functools.partial(_fused_body, cap=float(cap), t=T, kwe=KWE, kpe=KPE),
        grid=(KWE + C,),
        in_specs=[
            pl.BlockSpec((1, F, D),
                         lambda k: (jnp.minimum(k, KWE - 1) // KPE,
                                    jnp.minimum(k, KWE - 1) % KPE, 0)),
            pl.BlockSpec((T, D), lambda k: (jnp.maximum(k - KWE, 0), 0)),
            pl.BlockSpec((D, E), lambda k: (0, 0)),
            pl.BlockSpec((E, D), lambda k: (0, 0)),
            pl.BlockSpec((T, T), lambda k: (0, 0)),
        ],
        out_specs=pl.BlockSpec((T, 1), lambda k: (jnp.maximum(k - KWE, 0), 0)),
        out_shape=jax.ShapeDtypeStruct((S, 1), jnp.float32),
        scratch_shapes=[
            pltpu.VMEM((4, D), jnp.float32),
            pltpu.SMEM((1,), jnp.float32),
        ],
    )(We, x, wg, be, sl)

    v = val.reshape(B, SEQ)

    out = pl.pallas_call(
        _lsm_body,
        in_specs=[pl.BlockSpec((B, SEQ), lambda: (0, 0))],
        out_specs=pl.BlockSpec((B, SEQ), lambda: (0, 0)),
        out_shape=jax.ShapeDtypeStruct((B, SEQ), jnp.float32),
    )(v)
    return out
